# Initial kernel scaffold; baseline (speedup 1.0000x reference)
import jax
import jax.numpy as jnp
from jax.experimental import pallas as pl


def kernel(wrapped_phase, table_select_weight, table, hop_size):
    raise NotImplementedError("write your pallas kernel here")


# speedup vs baseline: 4.5601x; 4.5601x over previous
import functools

import jax
import jax.numpy as jnp
from jax import lax
from jax.experimental import pallas as pl
from jax.experimental.pallas import tpu as pltpu
from jax.experimental.pallas import tpu_sc as plsc

_NUM_CORES = 2
_NUM_SUBCORES = 16
_LANES = 16
_CHUNK = 8192


@functools.partial(jax.jit, static_argnums=(4, 5, 6))
def _run(phase_flat, tsw_pad_flat, table, p2, batch, seq_len, tsw_w):
    num_tables, table_length = table.shape
    hop = p2.shape[0]
    col_mask = table_length - 1
    frames_per_chunk = _CHUNK // hop
    n_chunks = seq_len // _CHUNK
    n_workers = _NUM_CORES * _NUM_SUBCORES
    row_scale = float(num_tables - 1)
    vecs_per_frame = hop // _LANES

    mesh = plsc.VectorSubcoreMesh(
        core_axis_name="c", subcore_axis_name="s",
        num_cores=_NUM_CORES, num_subcores=_NUM_SUBCORES)

    @functools.partial(
        pl.kernel,
        out_type=jax.ShapeDtypeStruct((batch * seq_len,), jnp.float32),
        mesh=mesh,
        scratch_types=[
            pltpu.VMEM((num_tables, table_length), jnp.float32),
            pltpu.VMEM((tsw_w,), jnp.float32),
            pltpu.VMEM((tsw_w,), jnp.int32),
            pltpu.VMEM((tsw_w,), jnp.float32),
            pltpu.VMEM((hop,), jnp.float32),
            pltpu.VMEM((_CHUNK,), jnp.float32),
            pltpu.VMEM((_CHUNK,), jnp.float32),
        ],
        compiler_params=pltpu.CompilerParams(needs_layout_passes=False),
    )
    def run(phase_hbm, tsw_hbm, table_hbm, p2_hbm, out_hbm,
            tab_v, tsw_v, row_v, frac_v, p2_v, ph_v, out_v):
        wid = lax.axis_index("s") * _NUM_CORES + lax.axis_index("c")

        pltpu.sync_copy(table_hbm, tab_v)
        pltpu.sync_copy(p2_hbm, p2_v)
        pltpu.sync_copy(tsw_hbm.at[pl.ds(wid * tsw_w, tsw_w)], tsw_v)

        for j in range(tsw_w // _LANES):
            sl = pl.ds(j * _LANES, _LANES)
            w = tsw_v[sl] * row_scale
            r = jnp.clip(w.astype(jnp.int32), 0, num_tables - 2)
            row_v[sl] = r
            frac_v[sl] = w - r.astype(jnp.float32)

        row_base = wid * seq_len

        def chunk_body(c, carry):
            pltpu.sync_copy(phase_hbm.at[pl.ds(row_base + c * _CHUNK, _CHUNK)],
                            ph_v)

            def frame_body(fl, carry2):
                f = c * frames_per_chunk + fl
                fvec = jnp.full((_LANES,), f, dtype=jnp.int32)
                rf = plsc.load_gather(row_v, [fvec])
                qf = plsc.load_gather(frac_v, [fvec])
                rc = plsc.load_gather(row_v, [fvec + 1])
                qc = plsc.load_gather(frac_v, [fvec + 1])
                omqf = 1.0 - qf
                omqc = 1.0 - qc
                rf1 = rf + 1
                rc1 = rc + 1
                base = fl * hop
                for k in range(vecs_per_frame):
                    sl = pl.ds(base + k * _LANES, _LANES)
                    x = ph_v[sl] * float(table_length)
                    i0 = jnp.clip(x.astype(jnp.int32), 0, table_length - 1)
                    px = x - i0.astype(jnp.float32)
                    ompx = 1.0 - px
                    i1 = jnp.bitwise_and(i0 + 1, col_mask)
                    a00 = plsc.load_gather(tab_v, [rf, i0])
                    a01 = plsc.load_gather(tab_v, [rf, i1])
                    a10 = plsc.load_gather(tab_v, [rf1, i0])
                    a11 = plsc.load_gather(tab_v, [rf1, i1])
                    b00 = plsc.load_gather(tab_v, [rc, i0])
                    b01 = plsc.load_gather(tab_v, [rc, i1])
                    b10 = plsc.load_gather(tab_v, [rc1, i0])
                    b11 = plsc.load_gather(tab_v, [rc1, i1])
                    f0 = a00 * omqf + a10 * qf
                    f1 = a01 * omqf + a11 * qf
                    c0 = b00 * omqc + b10 * qc
                    c1 = b01 * omqc + b11 * qc
                    sf = f0 * ompx + f1 * px
                    sc = c0 * ompx + c1 * px
                    p2k = p2_v[pl.ds(k * _LANES, _LANES)]
                    out_v[sl] = sf * (1.0 - p2k) + sc * p2k
                return carry2

            lax.fori_loop(0, frames_per_chunk, frame_body, 0)
            pltpu.sync_copy(out_v,
                            out_hbm.at[pl.ds(row_base + c * _CHUNK, _CHUNK)])
            return carry

        lax.fori_loop(0, n_chunks, chunk_body, 0)

    return run(phase_flat, tsw_pad_flat, table, p2)


def kernel(wrapped_phase, table_select_weight, table, hop_size):
    batch, seq_len = wrapped_phase.shape
    n_frames_p1 = table_select_weight.shape[1]
    hop = seq_len // (n_frames_p1 - 1)

    assert batch == _NUM_CORES * _NUM_SUBCORES
    assert hop % _LANES == 0 and _CHUNK % hop == 0 and seq_len % _CHUNK == 0
    assert table.shape[1] & (table.shape[1] - 1) == 0

    tsw_w = -(-n_frames_p1 // _LANES) * _LANES
    tsw_pad = jnp.pad(table_select_weight, ((0, 0), (0, tsw_w - n_frames_p1)))
    p2 = jnp.arange(hop, dtype=wrapped_phase.dtype) / hop_size

    out = _run(wrapped_phase.reshape(-1), tsw_pad.reshape(-1),
               table, p2, batch, seq_len, tsw_w)
    return out.reshape(batch, seq_len)


# speedup vs baseline: 4.9869x; 1.0936x over previous
import functools

import jax
import jax.numpy as jnp
from jax import lax
from jax.experimental import pallas as pl
from jax.experimental.pallas import tpu as pltpu
from jax.experimental.pallas import tpu_sc as plsc

_NUM_CORES = 2
_NUM_SUBCORES = 16
_LANES = 16
_CHUNK = 8192


@functools.partial(jax.jit, static_argnums=(5, 6, 7))
def _run(phase_flat, tsw_pad_flat, packed_table, p2, omp2,
         batch, seq_len, tsw_w):
    num_tables, table_length = packed_table.shape
    hop = p2.shape[0]
    frames_per_chunk = _CHUNK // hop
    n_chunks = seq_len // _CHUNK
    row_scale = float(num_tables - 1)
    vecs_per_frame = hop // _LANES

    mesh = plsc.VectorSubcoreMesh(
        core_axis_name="c", subcore_axis_name="s",
        num_cores=_NUM_CORES, num_subcores=_NUM_SUBCORES)

    @functools.partial(
        pl.kernel,
        out_type=jax.ShapeDtypeStruct((batch * seq_len,), jnp.float32),
        mesh=mesh,
        scratch_types=[
            pltpu.VMEM((num_tables, table_length), jnp.int32),
            pltpu.VMEM((tsw_w,), jnp.float32),
            pltpu.VMEM((tsw_w,), jnp.int32),
            pltpu.VMEM((tsw_w,), jnp.float32),
            pltpu.VMEM((hop,), jnp.float32),
            pltpu.VMEM((hop,), jnp.float32),
            pltpu.VMEM((_CHUNK,), jnp.float32),
            pltpu.VMEM((_CHUNK,), jnp.float32),
        ],
        compiler_params=pltpu.CompilerParams(needs_layout_passes=False),
    )
    def run(phase_hbm, tsw_hbm, table_hbm, p2_hbm, omp2_hbm, out_hbm,
            tab_v, tsw_v, row_v, frac_v, p2_v, omp2_v, ph_v, out_v):
        wid = lax.axis_index("s") * _NUM_CORES + lax.axis_index("c")

        pltpu.sync_copy(table_hbm, tab_v)
        pltpu.sync_copy(p2_hbm, p2_v)
        pltpu.sync_copy(omp2_hbm, omp2_v)
        pltpu.sync_copy(tsw_hbm.at[pl.ds(wid * tsw_w, tsw_w)], tsw_v)

        for j in range(tsw_w // _LANES):
            sl = pl.ds(j * _LANES, _LANES)
            w = tsw_v[sl] * row_scale
            r = jnp.clip(w.astype(jnp.int32), 0, num_tables - 2)
            row_v[sl] = r
            frac_v[sl] = w - r.astype(jnp.float32)

        row_base = wid * seq_len

        def chunk_body(c, carry):
            pltpu.sync_copy(phase_hbm.at[pl.ds(row_base + c * _CHUNK, _CHUNK)],
                            ph_v)

            def frame_body(fl, carry2):
                f = c * frames_per_chunk + fl
                fvec = jnp.full((_LANES,), f, dtype=jnp.int32)
                rf = plsc.load_gather(row_v, [fvec])
                qf = plsc.load_gather(frac_v, [fvec])
                rc = plsc.load_gather(row_v, [fvec + 1])
                qc = plsc.load_gather(frac_v, [fvec + 1])
                omqf = 1.0 - qf
                omqc = 1.0 - qc
                rf1 = rf + 1
                rc1 = rc + 1
                base = fl * hop
                for k in range(vecs_per_frame):
                    sl = pl.ds(base + k * _LANES, _LANES)
                    ksl = pl.ds(k * _LANES, _LANES)
                    x = ph_v[sl] * float(table_length)
                    i0 = jnp.clip(x.astype(jnp.int32), 0, table_length - 1)
                    px = x - i0.astype(jnp.float32)
                    ompx = 1.0 - px
                    p2k = p2_v[ksl]
                    omp2k = omp2_v[ksl]
                    w00 = plsc.pack(omp2k * omqf, omp2k * omqf,
                                    format=plsc.PackFormat.INTERLEAVED)
                    w01 = plsc.pack(omp2k * qf, omp2k * qf,
                                    format=plsc.PackFormat.INTERLEAVED)
                    w10 = plsc.pack(p2k * omqc, p2k * omqc,
                                    format=plsc.PackFormat.INTERLEAVED)
                    w11 = plsc.pack(p2k * qc, p2k * qc,
                                    format=plsc.PackFormat.INTERLEAVED)
                    g0 = plsc.load_gather(tab_v, [rf, i0])
                    g1 = plsc.load_gather(tab_v, [rf1, i0])
                    g2 = plsc.load_gather(tab_v, [rc, i0])
                    g3 = plsc.load_gather(tab_v, [rc1, i0])
                    p0 = plsc.bitcast(g0, jnp.bfloat16)
                    p1 = plsc.bitcast(g1, jnp.bfloat16)
                    p2_ = plsc.bitcast(g2, jnp.bfloat16)
                    p3 = plsc.bitcast(g3, jnp.bfloat16)
                    acc = p0 * w00 + p1 * w01 + p2_ * w10 + p3 * w11
                    u, v = plsc.unpack(acc, format=plsc.PackFormat.INTERLEAVED)
                    out_v[sl] = u * ompx + v * px
                return carry2

            lax.fori_loop(0, frames_per_chunk, frame_body, 0)
            pltpu.sync_copy(out_v,
                            out_hbm.at[pl.ds(row_base + c * _CHUNK, _CHUNK)])
            return carry

        lax.fori_loop(0, n_chunks, chunk_body, 0)

    return run(phase_flat, tsw_pad_flat, packed_table, p2, omp2)


def kernel(wrapped_phase, table_select_weight, table, hop_size):
    batch, seq_len = wrapped_phase.shape
    n_frames_p1 = table_select_weight.shape[1]
    hop = seq_len // (n_frames_p1 - 1)

    assert batch == _NUM_CORES * _NUM_SUBCORES
    assert hop % _LANES == 0 and _CHUNK % hop == 0 and seq_len % _CHUNK == 0

    tb = table.astype(jnp.bfloat16)
    tb1 = jnp.roll(tb, -1, axis=1)
    lo = jax.lax.bitcast_convert_type(tb, jnp.uint16).astype(jnp.uint32)
    hi = jax.lax.bitcast_convert_type(tb1, jnp.uint16).astype(jnp.uint32)
    packed_table = jax.lax.bitcast_convert_type(lo | (hi << 16), jnp.int32)

    tsw_w = -(-n_frames_p1 // _LANES) * _LANES
    tsw_pad = jnp.pad(table_select_weight, ((0, 0), (0, tsw_w - n_frames_p1)))
    p2 = jnp.arange(hop, dtype=wrapped_phase.dtype) / hop_size
    omp2 = 1.0 - p2

    out = _run(wrapped_phase.reshape(-1), tsw_pad.reshape(-1),
               packed_table, p2, omp2, batch, seq_len, tsw_w)
    return out.reshape(batch, seq_len)


# speedup vs baseline: 5.9031x; 1.1837x over previous
import functools

import jax
import jax.numpy as jnp
from jax import lax
from jax.experimental import pallas as pl
from jax.experimental.pallas import tpu as pltpu
from jax.experimental.pallas import tpu_sc as plsc

_NUM_CORES = 2
_NUM_SUBCORES = 16
_LANES = 16
_CHUNK = 4096


@functools.partial(jax.jit, static_argnums=(5,))
def _run(phase, tsw_pad, packed_flat, p2pack, omp2pack, hop):
    batch, seq_len = phase.shape
    tsw_w = tsw_pad.shape[1]
    flat_len = packed_flat.shape[0]

    frames_per_chunk = _CHUNK // hop
    n_pairs = seq_len // (2 * _CHUNK)
    vecs_per_frame = hop // _LANES

    mesh = plsc.VectorSubcoreMesh(
        core_axis_name="c", subcore_axis_name="s",
        num_cores=_NUM_CORES, num_subcores=_NUM_SUBCORES)

    @functools.partial(
        pl.kernel,
        out_type=jax.ShapeDtypeStruct((batch, seq_len), jnp.float32),
        mesh=mesh,
        scratch_types=[
            pltpu.VMEM((flat_len,), jnp.int32),
            pltpu.VMEM((tsw_w,), jnp.float32),
            pltpu.VMEM((tsw_w,), jnp.int32),
            pltpu.VMEM((tsw_w,), jnp.float32),
            pltpu.VMEM((hop,), jnp.int32),
            pltpu.VMEM((hop,), jnp.int32),
            pltpu.VMEM((_CHUNK,), jnp.float32),
            pltpu.VMEM((_CHUNK,), jnp.float32),
            pltpu.VMEM((_CHUNK,), jnp.float32),
            pltpu.VMEM((_CHUNK,), jnp.float32),
            pltpu.SemaphoreType.DMA,
            pltpu.SemaphoreType.DMA,
            pltpu.SemaphoreType.DMA,
            pltpu.SemaphoreType.DMA,
            pltpu.SemaphoreType.DMA,
        ],
        compiler_params=pltpu.CompilerParams(needs_layout_passes=False),
    )
    def run(phase_hbm, tsw_hbm, table_hbm, p2p_hbm, omp2p_hbm, out_hbm,
            tab_v, tsw_v, row_v, frac_v, p2p_v, omp2p_v,
            ph0_v, ph1_v, ou0_v, ou1_v, si0, si1, so0, so1, st):
        wid = lax.axis_index("s") * _NUM_CORES + lax.axis_index("c")

        table_dma = pltpu.async_copy(table_hbm, tab_v, st)
        pltpu.sync_copy(p2p_hbm, p2p_v)
        pltpu.sync_copy(omp2p_hbm, omp2p_v)
        pltpu.sync_copy(tsw_hbm.at[wid], tsw_v)

        t_minus_1 = float(_NUM_TABLES - 1)
        for j in range(tsw_w // _LANES):
            sl = pl.ds(j * _LANES, _LANES)
            w = tsw_v[sl] * t_minus_1
            r = jnp.clip(w.astype(jnp.int32), 0, _NUM_TABLES - 2)
            row_v[sl] = r * _TABLE_LEN
            frac_v[sl] = w - r.astype(jnp.float32)

        table_dma.wait()

        def compute(ph_v, out_v, c):
            def frame_body(fl, carry2):
                f = c * frames_per_chunk + fl
                fvec = jnp.full((_LANES,), f, dtype=jnp.int32)
                rfb = plsc.load_gather(row_v, [fvec])
                qf = plsc.load_gather(frac_v, [fvec])
                rcb = plsc.load_gather(row_v, [fvec + 1])
                qc = plsc.load_gather(frac_v, [fvec + 1])
                omqf = 1.0 - qf
                omqc = 1.0 - qc
                ifmt = plsc.PackFormat.INTERLEAVED
                wf0 = plsc.pack(omqf, omqf, format=ifmt)
                wf1 = plsc.pack(qf, qf, format=ifmt)
                wc0 = plsc.pack(omqc, omqc, format=ifmt)
                wc1 = plsc.pack(qc, qc, format=ifmt)
                rf1b = rfb + _TABLE_LEN
                rc1b = rcb + _TABLE_LEN
                base = fl * hop
                for k in range(vecs_per_frame):
                    sl = pl.ds(base + k * _LANES, _LANES)
                    ksl = pl.ds(k * _LANES, _LANES)
                    x = ph_v[sl] * float(_TABLE_LEN)
                    i0 = jnp.clip(x.astype(jnp.int32), 0, _TABLE_LEN - 1)
                    px = x - i0.astype(jnp.float32)
                    ompx = 1.0 - px
                    g0 = plsc.load_gather(tab_v, [rfb + i0])
                    g1 = plsc.load_gather(tab_v, [rf1b + i0])
                    g2 = plsc.load_gather(tab_v, [rcb + i0])
                    g3 = plsc.load_gather(tab_v, [rc1b + i0])
                    p0 = plsc.bitcast(g0, jnp.bfloat16)
                    p1 = plsc.bitcast(g1, jnp.bfloat16)
                    p2_ = plsc.bitcast(g2, jnp.bfloat16)
                    p3 = plsc.bitcast(g3, jnp.bfloat16)
                    sfp = p0 * wf0 + p1 * wf1
                    scp = p2_ * wc0 + p3 * wc1
                    p2k = plsc.bitcast(p2p_v[ksl], jnp.bfloat16)
                    omp2k = plsc.bitcast(omp2p_v[ksl], jnp.bfloat16)
                    acc = sfp * omp2k + scp * p2k
                    u, v = plsc.unpack(acc, format=ifmt)
                    out_v[sl] = u * ompx + v * px
                return carry2

            lax.fori_loop(0, frames_per_chunk, frame_body, 0)

        def start_in(buf, sem, c):
            pltpu.async_copy(phase_hbm.at[wid, pl.ds(c * _CHUNK, _CHUNK)],
                             buf, sem)

        def start_out(buf, sem, c):
            pltpu.async_copy(buf, out_hbm.at[wid, pl.ds(c * _CHUNK, _CHUNK)],
                             sem)

        def wait_in(buf, sem):
            pltpu.make_async_copy(phase_hbm.at[wid, pl.ds(0, _CHUNK)],
                                  buf, sem).wait()

        def wait_out(buf, sem):
            pltpu.make_async_copy(buf, out_hbm.at[wid, pl.ds(0, _CHUNK)],
                                  sem).wait()

        start_in(ph0_v, si0, 0)
        start_in(ph1_v, si1, 1)

        def pair_body(c2, carry):
            c0 = 2 * c2

            wait_in(ph0_v, si0)

            @pl.when(c2 > 0)
            def _():
                wait_out(ou0_v, so0)

            compute(ph0_v, ou0_v, c0)
            start_out(ou0_v, so0, c0)

            @pl.when(c2 < n_pairs - 1)
            def _():
                start_in(ph0_v, si0, c0 + 2)

            wait_in(ph1_v, si1)

            @pl.when(c2 > 0)
            def _():
                wait_out(ou1_v, so1)

            compute(ph1_v, ou1_v, c0 + 1)
            start_out(ou1_v, so1, c0 + 1)

            @pl.when(c2 < n_pairs - 1)
            def _():
                start_in(ph1_v, si1, c0 + 3)

            return carry

        lax.fori_loop(0, n_pairs, pair_body, 0)
        wait_out(ou0_v, so0)
        wait_out(ou1_v, so1)

    return run(phase, tsw_pad, packed_flat, p2pack, omp2pack)


_NUM_TABLES = 100
_TABLE_LEN = 1024


def _pack_pair_bits(a_bf16, b_bf16):
    lo = jax.lax.bitcast_convert_type(a_bf16, jnp.uint16).astype(jnp.uint32)
    hi = jax.lax.bitcast_convert_type(b_bf16, jnp.uint16).astype(jnp.uint32)
    return jax.lax.bitcast_convert_type(lo | (hi << 16), jnp.int32)


def kernel(wrapped_phase, table_select_weight, table, hop_size):
    batch, seq_len = wrapped_phase.shape
    n_frames_p1 = table_select_weight.shape[1]
    hop = seq_len // (n_frames_p1 - 1)

    assert batch == _NUM_CORES * _NUM_SUBCORES
    assert table.shape == (_NUM_TABLES, _TABLE_LEN)
    assert hop % _LANES == 0 and _CHUNK % hop == 0
    assert seq_len % (2 * _CHUNK) == 0

    tb = table.astype(jnp.bfloat16)
    packed_flat = _pack_pair_bits(tb, jnp.roll(tb, -1, axis=1)).reshape(-1)

    tsw_w = -(-n_frames_p1 // _LANES) * _LANES
    tsw_pad = jnp.pad(table_select_weight, ((0, 0), (0, tsw_w - n_frames_p1)))

    p2 = (jnp.arange(hop, dtype=wrapped_phase.dtype) / hop_size)
    p2pack = _pack_pair_bits(p2.astype(jnp.bfloat16), p2.astype(jnp.bfloat16))
    omp2 = (1.0 - p2).astype(jnp.bfloat16)
    omp2pack = _pack_pair_bits(omp2, omp2)

    return _run(wrapped_phase, tsw_pad, packed_flat, p2pack, omp2pack, hop)


# speedup vs baseline: 9.6327x; 1.6318x over previous
import functools

import jax
import jax.numpy as jnp
from jax import lax
from jax.experimental import pallas as pl
from jax.experimental.pallas import tpu as pltpu
from jax.experimental.pallas import tpu_sc as plsc

_NUM_CORES = 2
_NUM_SUBCORES = 16
_LANES = 16
_CHUNK = 4096


@functools.partial(jax.jit, static_argnums=(5,))
def _run(phase, tsw_pad, packed_flat, p2pack, omp2pack, hop):
    batch, seq_len = phase.shape
    tsw_w = tsw_pad.shape[1]
    flat_len = packed_flat.shape[0]

    frames_per_chunk = _CHUNK // hop
    n_pairs = seq_len // (2 * _CHUNK)
    vecs_per_frame = hop // _LANES

    mesh = plsc.VectorSubcoreMesh(
        core_axis_name="c", subcore_axis_name="s",
        num_cores=_NUM_CORES, num_subcores=_NUM_SUBCORES)

    @functools.partial(
        pl.kernel,
        out_type=jax.ShapeDtypeStruct((batch, seq_len), jnp.float32),
        mesh=mesh,
        scratch_types=[
            pltpu.VMEM((flat_len,), jnp.int32),
            pltpu.VMEM((tsw_w,), jnp.float32),
            pltpu.VMEM((tsw_w,), jnp.int32),
            pltpu.VMEM((tsw_w,), jnp.float32),
            pltpu.VMEM((hop,), jnp.int32),
            pltpu.VMEM((hop,), jnp.int32),
            pltpu.VMEM((_CHUNK,), jnp.float32),
            pltpu.VMEM((_CHUNK,), jnp.float32),
            pltpu.VMEM((_CHUNK,), jnp.float32),
            pltpu.VMEM((_CHUNK,), jnp.float32),
            pltpu.SemaphoreType.DMA,
            pltpu.SemaphoreType.DMA,
            pltpu.SemaphoreType.DMA,
            pltpu.SemaphoreType.DMA,
            pltpu.SemaphoreType.DMA,
        ],
        compiler_params=pltpu.CompilerParams(needs_layout_passes=False),
    )
    def run(phase_hbm, tsw_hbm, table_hbm, p2p_hbm, omp2p_hbm, out_hbm,
            tab_v, tsw_v, row_v, frac_v, p2p_v, omp2p_v,
            ph0_v, ph1_v, ou0_v, ou1_v, si0, si1, so0, so1, st):
        wid = lax.axis_index("s") * _NUM_CORES + lax.axis_index("c")

        table_dma = pltpu.async_copy(table_hbm, tab_v, st)
        pltpu.sync_copy(p2p_hbm, p2p_v)
        pltpu.sync_copy(omp2p_hbm, omp2p_v)
        pltpu.sync_copy(tsw_hbm.at[wid], tsw_v)

        t_minus_1 = float(_NUM_TABLES - 1)
        for j in range(tsw_w // _LANES):
            sl = pl.ds(j * _LANES, _LANES)
            w = tsw_v[sl] * t_minus_1
            r = jnp.clip(w.astype(jnp.int32), 0, _NUM_TABLES - 2)
            row_v[sl] = r * _TABLE_LEN
            frac_v[sl] = w - r.astype(jnp.float32)

        table_dma.wait()

        def compute(ph_v, out_v, c):
            def frame_body(fl, carry2):
                f = c * frames_per_chunk + fl
                fvec = jnp.full((_LANES,), f, dtype=jnp.int32)
                rfb = plsc.load_gather(row_v, [fvec])
                qf = plsc.load_gather(frac_v, [fvec])
                rcb = plsc.load_gather(row_v, [fvec + 1])
                qc = plsc.load_gather(frac_v, [fvec + 1])
                omqf = 1.0 - qf
                omqc = 1.0 - qc
                ifmt = plsc.PackFormat.INTERLEAVED
                wf0 = plsc.pack(omqf, omqf, format=ifmt)
                wf1 = plsc.pack(qf, qf, format=ifmt)
                wc0 = plsc.pack(omqc, omqc, format=ifmt)
                wc1 = plsc.pack(qc, qc, format=ifmt)
                rf1b = rfb + _TABLE_LEN
                rc1b = rcb + _TABLE_LEN
                base = fl * hop

                @plsc.parallel_loop(0, hop, _LANES, unroll=4)
                def _(k):
                    sl = pl.ds(base + k, _LANES)
                    ksl = pl.ds(k, _LANES)
                    x = ph_v[sl] * float(_TABLE_LEN)
                    i0 = jnp.clip(x.astype(jnp.int32), 0, _TABLE_LEN - 1)
                    px = x - i0.astype(jnp.float32)
                    ompx = 1.0 - px
                    g0 = plsc.load_gather(tab_v, [rfb + i0])
                    g1 = plsc.load_gather(tab_v, [rf1b + i0])
                    g2 = plsc.load_gather(tab_v, [rcb + i0])
                    g3 = plsc.load_gather(tab_v, [rc1b + i0])
                    p0 = plsc.bitcast(g0, jnp.bfloat16)
                    p1 = plsc.bitcast(g1, jnp.bfloat16)
                    p2_ = plsc.bitcast(g2, jnp.bfloat16)
                    p3 = plsc.bitcast(g3, jnp.bfloat16)
                    sfp = p0 * wf0 + p1 * wf1
                    scp = p2_ * wc0 + p3 * wc1
                    p2k = plsc.bitcast(p2p_v[ksl], jnp.bfloat16)
                    omp2k = plsc.bitcast(omp2p_v[ksl], jnp.bfloat16)
                    acc = sfp * omp2k + scp * p2k
                    u, v = plsc.unpack(acc, format=ifmt)
                    out_v[sl] = u * ompx + v * px

                return carry2

            lax.fori_loop(0, frames_per_chunk, frame_body, 0)

        def start_in(buf, sem, c):
            pltpu.async_copy(phase_hbm.at[wid, pl.ds(c * _CHUNK, _CHUNK)],
                             buf, sem)

        def start_out(buf, sem, c):
            pltpu.async_copy(buf, out_hbm.at[wid, pl.ds(c * _CHUNK, _CHUNK)],
                             sem)

        def wait_in(buf, sem):
            pltpu.make_async_copy(phase_hbm.at[wid, pl.ds(0, _CHUNK)],
                                  buf, sem).wait()

        def wait_out(buf, sem):
            pltpu.make_async_copy(buf, out_hbm.at[wid, pl.ds(0, _CHUNK)],
                                  sem).wait()

        start_in(ph0_v, si0, 0)
        start_in(ph1_v, si1, 1)

        def pair_body(c2, carry):
            c0 = 2 * c2

            wait_in(ph0_v, si0)

            @pl.when(c2 > 0)
            def _():
                wait_out(ou0_v, so0)

            compute(ph0_v, ou0_v, c0)
            start_out(ou0_v, so0, c0)

            @pl.when(c2 < n_pairs - 1)
            def _():
                start_in(ph0_v, si0, c0 + 2)

            wait_in(ph1_v, si1)

            @pl.when(c2 > 0)
            def _():
                wait_out(ou1_v, so1)

            compute(ph1_v, ou1_v, c0 + 1)
            start_out(ou1_v, so1, c0 + 1)

            @pl.when(c2 < n_pairs - 1)
            def _():
                start_in(ph1_v, si1, c0 + 3)

            return carry

        lax.fori_loop(0, n_pairs, pair_body, 0)
        wait_out(ou0_v, so0)
        wait_out(ou1_v, so1)

    return run(phase, tsw_pad, packed_flat, p2pack, omp2pack)


_NUM_TABLES = 100
_TABLE_LEN = 1024


def _pack_pair_bits(a_bf16, b_bf16):
    lo = jax.lax.bitcast_convert_type(a_bf16, jnp.uint16).astype(jnp.uint32)
    hi = jax.lax.bitcast_convert_type(b_bf16, jnp.uint16).astype(jnp.uint32)
    return jax.lax.bitcast_convert_type(lo | (hi << 16), jnp.int32)


def kernel(wrapped_phase, table_select_weight, table, hop_size):
    batch, seq_len = wrapped_phase.shape
    n_frames_p1 = table_select_weight.shape[1]
    hop = seq_len // (n_frames_p1 - 1)

    assert batch == _NUM_CORES * _NUM_SUBCORES
    assert table.shape == (_NUM_TABLES, _TABLE_LEN)
    assert hop % _LANES == 0 and _CHUNK % hop == 0
    assert seq_len % (2 * _CHUNK) == 0

    tb = table.astype(jnp.bfloat16)
    packed_flat = _pack_pair_bits(tb, jnp.roll(tb, -1, axis=1)).reshape(-1)

    tsw_w = -(-n_frames_p1 // _LANES) * _LANES
    tsw_pad = jnp.pad(table_select_weight, ((0, 0), (0, tsw_w - n_frames_p1)))

    p2 = (jnp.arange(hop, dtype=wrapped_phase.dtype) / hop_size)
    p2pack = _pack_pair_bits(p2.astype(jnp.bfloat16), p2.astype(jnp.bfloat16))
    omp2 = (1.0 - p2).astype(jnp.bfloat16)
    omp2pack = _pack_pair_bits(omp2, omp2)

    return _run(wrapped_phase, tsw_pad, packed_flat, p2pack, omp2pack, hop)
